# baseline (speedup 1.0000x reference)
import jax
import jax.numpy as jnp
from jax import lax
from jax.experimental import pallas as pl
from jax.experimental.pallas import tpu as pltpu
from jax.experimental.pallas import tpu_sc as plsc

_B, _S, _D = 4, 4160, 1024
_L = 13
_G = _S // _L
_NP = 8
_DH = 512
_LANES = 16
_NW = 32
_UNITS = _B * (_G // _NP) * (_D // _DH)
_UPW = _UNITS // _NW
_CPD = _DH // _LANES

_NET = (
    (0, 1), (2, 3), (4, 5), (6, 7), (8, 9), (10, 11),
    (0, 2), (1, 3), (4, 6), (5, 7), (8, 10), (9, 11),
    (1, 2), (5, 6), (9, 10),
    (0, 4), (1, 5), (2, 6), (3, 7), (8, 12),
    (2, 4), (3, 5), (10, 12),
    (1, 2), (3, 4), (5, 6), (9, 10), (11, 12),
    (0, 8), (1, 9), (2, 10), (3, 11), (4, 12),
    (4, 8), (5, 9), (6, 10), (7, 11),
    (2, 4), (3, 5), (6, 8), (7, 9), (10, 12),
    (1, 2), (3, 4), (5, 6), (7, 8), (9, 10), (11, 12),
)


def _net_sorted(xs):
    xs = list(xs)
    for a, b in _NET:
        lo = jnp.minimum(xs[a], xs[b])
        hi = jnp.maximum(xs[a], xs[b])
        xs[a] = lo
        xs[b] = hi
    return xs


def _sort_window(buf, is_w0):
    @pl.when(jnp.logical_not(is_w0))
    def _():
        @plsc.parallel_loop(0, _NP * _CPD, unroll=4)
        def _body(i):
            p = i >> 5
            off = (i & (_CPD - 1)) * _LANES
            ys = _net_sorted([buf[t, p, pl.ds(off, _LANES)]
                              for t in range(_L)])
            for k in range(_L):
                buf[k, p, pl.ds(off, _LANES)] = ys[k]

    @pl.when(is_w0)
    def _():
        @plsc.parallel_loop(0, (_NP - 1) * _CPD, unroll=4)
        def _body_rot(i):
            p = i >> 5
            off = (i & (_CPD - 1)) * _LANES
            ys = _net_sorted([buf[t, p, pl.ds(off, _LANES)]
                              for t in range(_L)])
            for k in range(_L):
                buf[(k + 1) % _L, p, pl.ds(off, _LANES)] = ys[k]

        @plsc.parallel_loop(0, _CPD, unroll=4)
        def _body_p7(c):
            off = c * _LANES
            ys = _net_sorted([buf[t, _NP - 1, pl.ds(off, _LANES)]
                              for t in range(_L)])
            for k in range(_L):
                buf[k, _NP - 1, pl.ds(off, _LANES)] = ys[k]


def _sc_body(v_hbm, out_hbm, buf_a, buf_b, sa_i, sb_i, sa_o, sb_o):
    wid = lax.axis_index("s") * 2 + lax.axis_index("c")

    def unit(hbm, g):
        b = g // (_UNITS // _B)
        r = g % (_UNITS // _B)
        w = r >> 1
        r0 = w * _NP
        dc0 = (r & 1) * _DH
        return hbm.at[b, :, pl.ds(r0, _NP), pl.ds(dc0, _DH)], w == 0

    def pair_body(p, carry):
        ga = wid * _UPW + 2 * p
        gb = ga + 1
        src_a, w0_a = unit(v_hbm, ga)
        src_b, w0_b = unit(v_hbm, gb)
        ha = pltpu.async_copy(src_a, buf_a, sa_i)
        hb = pltpu.async_copy(src_b, buf_b, sb_i)
        ha.wait()
        _sort_window(buf_a, w0_a)
        dst_a, _ = unit(out_hbm, ga)
        oa = pltpu.async_copy(buf_a, dst_a, sa_o)
        hb.wait()
        _sort_window(buf_b, w0_b)
        dst_b, _ = unit(out_hbm, gb)
        ob = pltpu.async_copy(buf_b, dst_b, sb_o)
        oa.wait()
        ob.wait()
        return carry

    lax.fori_loop(0, _UPW // 2, pair_body, 0)


_sc_sort = pl.kernel(
    _sc_body,
    out_type=jax.ShapeDtypeStruct((_B, _L, _G, _D), jnp.float32),
    mesh=plsc.VectorSubcoreMesh(core_axis_name="c", subcore_axis_name="s"),
    scratch_types=[
        pltpu.VMEM((_L, _NP, _DH), jnp.float32),
        pltpu.VMEM((_L, _NP, _DH), jnp.float32),
        pltpu.SemaphoreType.DMA,
        pltpu.SemaphoreType.DMA,
        pltpu.SemaphoreType.DMA,
        pltpu.SemaphoreType.DMA,
    ],
)


def kernel(q, k, v):
    del q, k
    out = _sc_sort(v.reshape(_B, _L, _G, _D))
    return out.reshape(_B, _S, _D)


# speedup vs baseline: 1.1847x; 1.1847x over previous
import jax
import jax.numpy as jnp
from jax import lax
from jax.experimental import pallas as pl
from jax.experimental.pallas import tpu as pltpu
from jax.experimental.pallas import tpu_sc as plsc

_B, _S, _D = 4, 4160, 1024
_L = 13
_G = _S // _L
_NP = 8
_DH = 512
_LANES = 16
_NW = 32
_UNITS = _B * (_G // _NP) * (_D // _DH)
_UPW = _UNITS // _NW
_CPD = _DH // _LANES

_NET = (
    (0, 1), (2, 3), (4, 5), (6, 7), (8, 9), (10, 11),
    (0, 2), (1, 3), (4, 6), (5, 7), (8, 10), (9, 11),
    (1, 2), (5, 6), (9, 10),
    (0, 4), (1, 5), (2, 6), (3, 7), (8, 12),
    (2, 4), (3, 5), (10, 12),
    (1, 2), (3, 4), (5, 6), (9, 10), (11, 12),
    (0, 8), (1, 9), (2, 10), (3, 11), (4, 12),
    (4, 8), (5, 9), (6, 10), (7, 11),
    (2, 4), (3, 5), (6, 8), (7, 9), (10, 12),
    (1, 2), (3, 4), (5, 6), (7, 8), (9, 10), (11, 12),
)


def _net_sorted(xs):
    xs = list(xs)
    for a, b in _NET:
        lo = jnp.minimum(xs[a], xs[b])
        hi = jnp.maximum(xs[a], xs[b])
        xs[a] = lo
        xs[b] = hi
    return xs


def _sort_window(buf, is_w0):
    @pl.when(jnp.logical_not(is_w0))
    def _():
        @plsc.parallel_loop(0, _NP * _CPD, unroll=2)
        def _body(i):
            p = i >> 5
            off = (i & (_CPD - 1)) * _LANES
            ys = _net_sorted([buf[t, p, pl.ds(off, _LANES)]
                              for t in range(_L)])
            for k in range(_L):
                buf[k, p, pl.ds(off, _LANES)] = ys[k]

    @pl.when(is_w0)
    def _():
        @plsc.parallel_loop(0, (_NP - 1) * _CPD, unroll=2)
        def _body_rot(i):
            p = i >> 5
            off = (i & (_CPD - 1)) * _LANES
            ys = _net_sorted([buf[t, p, pl.ds(off, _LANES)]
                              for t in range(_L)])
            for k in range(_L):
                buf[(k + 1) % _L, p, pl.ds(off, _LANES)] = ys[k]

        @plsc.parallel_loop(0, _CPD, unroll=2)
        def _body_p7(c):
            off = c * _LANES
            ys = _net_sorted([buf[t, _NP - 1, pl.ds(off, _LANES)]
                              for t in range(_L)])
            for k in range(_L):
                buf[k, _NP - 1, pl.ds(off, _LANES)] = ys[k]


def _sc_body(v_hbm, out_hbm, buf_a, buf_b, sa_i, sb_i, sa_o, sb_o):
    wid = lax.axis_index("s") * 2 + lax.axis_index("c")

    def unit(hbm, g):
        b = g // (_UNITS // _B)
        r = g % (_UNITS // _B)
        w = r >> 1
        r0 = w * _NP
        dc0 = (r & 1) * _DH
        return hbm.at[b, :, pl.ds(r0, _NP), pl.ds(dc0, _DH)], w == 0

    def pair_body(p, carry):
        ga = wid * _UPW + 2 * p
        gb = ga + 1
        src_a, w0_a = unit(v_hbm, ga)
        src_b, w0_b = unit(v_hbm, gb)
        ha = pltpu.async_copy(src_a, buf_a, sa_i)
        hb = pltpu.async_copy(src_b, buf_b, sb_i)
        ha.wait()
        _sort_window(buf_a, w0_a)
        dst_a, _ = unit(out_hbm, ga)
        oa = pltpu.async_copy(buf_a, dst_a, sa_o)
        hb.wait()
        _sort_window(buf_b, w0_b)
        dst_b, _ = unit(out_hbm, gb)
        ob = pltpu.async_copy(buf_b, dst_b, sb_o)
        oa.wait()
        ob.wait()
        return carry

    lax.fori_loop(0, _UPW // 2, pair_body, 0)


_sc_sort = pl.kernel(
    _sc_body,
    out_type=jax.ShapeDtypeStruct((_B, _L, _G, _D), jnp.float32),
    mesh=plsc.VectorSubcoreMesh(core_axis_name="c", subcore_axis_name="s"),
    scratch_types=[
        pltpu.VMEM((_L, _NP, _DH), jnp.float32),
        pltpu.VMEM((_L, _NP, _DH), jnp.float32),
        pltpu.SemaphoreType.DMA,
        pltpu.SemaphoreType.DMA,
        pltpu.SemaphoreType.DMA,
        pltpu.SemaphoreType.DMA,
    ],
)


def kernel(q, k, v):
    del q, k
    out = _sc_sort(v.reshape(_B, _L, _G, _D))
    return out.reshape(_B, _S, _D)


# speedup vs baseline: 1.2830x; 1.0830x over previous
import jax
import jax.numpy as jnp
from jax import lax
from jax.experimental import pallas as pl
from jax.experimental.pallas import tpu as pltpu
from jax.experimental.pallas import tpu_sc as plsc

_B, _S, _D = 4, 4160, 1024
_L = 13
_G = _S // _L
_NP = 8
_DH = 256
_LANES = 16
_NW = 32
_UNITS = _B * (_G // _NP) * (_D // _DH)
_UPW = _UNITS // _NW
_CPD = _DH // _LANES

_NET = (
    (0, 1), (2, 3), (4, 5), (6, 7), (8, 9), (10, 11),
    (0, 2), (1, 3), (4, 6), (5, 7), (8, 10), (9, 11),
    (1, 2), (5, 6), (9, 10),
    (0, 4), (1, 5), (2, 6), (3, 7), (8, 12),
    (2, 4), (3, 5), (10, 12),
    (1, 2), (3, 4), (5, 6), (9, 10), (11, 12),
    (0, 8), (1, 9), (2, 10), (3, 11), (4, 12),
    (4, 8), (5, 9), (6, 10), (7, 11),
    (2, 4), (3, 5), (6, 8), (7, 9), (10, 12),
    (1, 2), (3, 4), (5, 6), (7, 8), (9, 10), (11, 12),
)


def _net_sorted(xs):
    xs = list(xs)
    for a, b in _NET:
        lo = jnp.minimum(xs[a], xs[b])
        hi = jnp.maximum(xs[a], xs[b])
        xs[a] = lo
        xs[b] = hi
    return xs


def _sort_window(src, dst, is_w0):
    @pl.when(jnp.logical_not(is_w0))
    def _():
        @plsc.parallel_loop(0, _NP * _CPD, unroll=2)
        def _body(i):
            p = i // _CPD
            off = (i % _CPD) * _LANES
            ys = _net_sorted([src[t, p, pl.ds(off, _LANES)]
                              for t in range(_L)])
            for k in range(_L):
                dst[k, p, pl.ds(off, _LANES)] = ys[k]

    @pl.when(is_w0)
    def _():
        @plsc.parallel_loop(0, (_NP - 1) * _CPD, unroll=2)
        def _body_rot(i):
            p = i // _CPD
            off = (i % _CPD) * _LANES
            ys = _net_sorted([src[t, p, pl.ds(off, _LANES)]
                              for t in range(_L)])
            for k in range(_L):
                dst[(k + 1) % _L, p, pl.ds(off, _LANES)] = ys[k]

        @plsc.parallel_loop(0, _CPD, unroll=2)
        def _body_p7(c):
            off = c * _LANES
            ys = _net_sorted([src[t, _NP - 1, pl.ds(off, _LANES)]
                              for t in range(_L)])
            for k in range(_L):
                dst[k, _NP - 1, pl.ds(off, _LANES)] = ys[k]


def _sc_body(v_hbm, out_hbm, in_a, out_a, in_b, out_b,
             sa_i, sb_i, sa_o, sb_o):
    wid = lax.axis_index("s") * 2 + lax.axis_index("c")

    def unit(hbm, g):
        b = g // (_UNITS // _B)
        r = g % (_UNITS // _B)
        w = r // (_D // _DH)
        r0 = w * _NP
        dc0 = (r % (_D // _DH)) * _DH
        return hbm.at[b, :, pl.ds(r0, _NP), pl.ds(dc0, _DH)], w == 0

    def pair_body(p, carry):
        ga = wid * _UPW + 2 * p
        gb = ga + 1
        src_a, w0_a = unit(v_hbm, ga)
        src_b, w0_b = unit(v_hbm, gb)
        ha = pltpu.async_copy(src_a, in_a, sa_i)
        hb = pltpu.async_copy(src_b, in_b, sb_i)
        ha.wait()
        _sort_window(in_a, out_a, w0_a)
        dst_a, _ = unit(out_hbm, ga)
        oa = pltpu.async_copy(out_a, dst_a, sa_o)
        hb.wait()
        _sort_window(in_b, out_b, w0_b)
        dst_b, _ = unit(out_hbm, gb)
        ob = pltpu.async_copy(out_b, dst_b, sb_o)
        oa.wait()
        ob.wait()
        return carry

    lax.fori_loop(0, _UPW // 2, pair_body, 0)


_sc_sort = pl.kernel(
    _sc_body,
    out_type=jax.ShapeDtypeStruct((_B, _L, _G, _D), jnp.float32),
    mesh=plsc.VectorSubcoreMesh(core_axis_name="c", subcore_axis_name="s"),
    scratch_types=[
        pltpu.VMEM((_L, _NP, _DH), jnp.float32),
        pltpu.VMEM((_L, _NP, _DH), jnp.float32),
        pltpu.VMEM((_L, _NP, _DH), jnp.float32),
        pltpu.VMEM((_L, _NP, _DH), jnp.float32),
        pltpu.SemaphoreType.DMA,
        pltpu.SemaphoreType.DMA,
        pltpu.SemaphoreType.DMA,
        pltpu.SemaphoreType.DMA,
    ],
)


def kernel(q, k, v):
    del q, k
    out = _sc_sort(v.reshape(_B, _L, _G, _D))
    return out.reshape(_B, _S, _D)


# speedup vs baseline: 1.3012x; 1.0142x over previous
import jax
import jax.numpy as jnp
from jax import lax
from jax.experimental import pallas as pl
from jax.experimental.pallas import tpu as pltpu
from jax.experimental.pallas import tpu_sc as plsc

_B, _S, _D = 4, 4160, 1024
_L = 13
_G = _S // _L
_NP = 8
_DH = 128
_LANES = 16
_NW = 32
_UNITS = _B * (_G // _NP) * (_D // _DH)
_UPW = _UNITS // _NW
_CPD = _DH // _LANES

_NET = (
    (0, 1), (2, 3), (4, 5), (6, 7), (8, 9), (10, 11),
    (0, 2), (1, 3), (4, 6), (5, 7), (8, 10), (9, 11),
    (1, 2), (5, 6), (9, 10),
    (0, 4), (1, 5), (2, 6), (3, 7), (8, 12),
    (2, 4), (3, 5), (10, 12),
    (1, 2), (3, 4), (5, 6), (9, 10), (11, 12),
    (0, 8), (1, 9), (2, 10), (3, 11), (4, 12),
    (4, 8), (5, 9), (6, 10), (7, 11),
    (2, 4), (3, 5), (6, 8), (7, 9), (10, 12),
    (1, 2), (3, 4), (5, 6), (7, 8), (9, 10), (11, 12),
)


def _net_sorted(xs):
    xs = list(xs)
    for a, b in _NET:
        lo = jnp.minimum(xs[a], xs[b])
        hi = jnp.maximum(xs[a], xs[b])
        xs[a] = lo
        xs[b] = hi
    return xs


def _sort_window(src, dst, is_w0):
    @pl.when(jnp.logical_not(is_w0))
    def _():
        @plsc.parallel_loop(0, _NP * _CPD, unroll=2)
        def _body(i):
            p = i // _CPD
            off = (i % _CPD) * _LANES
            ys = _net_sorted([src[t, p, pl.ds(off, _LANES)]
                              for t in range(_L)])
            for k in range(_L):
                dst[k, p, pl.ds(off, _LANES)] = ys[k]

    @pl.when(is_w0)
    def _():
        @plsc.parallel_loop(0, (_NP - 1) * _CPD, unroll=2)
        def _body_rot(i):
            p = i // _CPD
            off = (i % _CPD) * _LANES
            ys = _net_sorted([src[t, p, pl.ds(off, _LANES)]
                              for t in range(_L)])
            for k in range(_L):
                dst[(k + 1) % _L, p, pl.ds(off, _LANES)] = ys[k]

        @plsc.parallel_loop(0, _CPD, unroll=2)
        def _body_p7(c):
            off = c * _LANES
            ys = _net_sorted([src[t, _NP - 1, pl.ds(off, _LANES)]
                              for t in range(_L)])
            for k in range(_L):
                dst[k, _NP - 1, pl.ds(off, _LANES)] = ys[k]


def _sc_body(v_hbm, out_hbm, in_a, out_a, in_b, out_b, in_c, out_c,
             in_d, out_d, sa_i, sb_i, sc_i, sd_i, sa_o, sb_o, sc_o, sd_o):
    wid = lax.axis_index("s") * 2 + lax.axis_index("c")

    def unit(hbm, g):
        b = g // (_UNITS // _B)
        r = g % (_UNITS // _B)
        w = r // (_D // _DH)
        r0 = w * _NP
        dc0 = (r % (_D // _DH)) * _DH
        return hbm.at[b, :, pl.ds(r0, _NP), pl.ds(dc0, _DH)], w == 0

    lanes = ((in_a, out_a, sa_i, sa_o), (in_b, out_b, sb_i, sb_o),
             (in_c, out_c, sc_i, sc_o), (in_d, out_d, sd_i, sd_o))

    def quad_body(p, carry):
        g0 = wid * _UPW + 4 * p
        hs = []
        for q, (ibuf, obuf, si, so) in enumerate(lanes):
            src, _ = unit(v_hbm, g0 + q)
            hs.append(pltpu.async_copy(src, ibuf, si))
        outs = []
        for q, (ibuf, obuf, si, so) in enumerate(lanes):
            hs[q].wait()
            _, w0 = unit(v_hbm, g0 + q)
            _sort_window(ibuf, obuf, w0)
            dst, _ = unit(out_hbm, g0 + q)
            outs.append(pltpu.async_copy(obuf, dst, so))
        for o in outs:
            o.wait()
        return carry

    lax.fori_loop(0, _UPW // 4, quad_body, 0)


_sc_sort = pl.kernel(
    _sc_body,
    out_type=jax.ShapeDtypeStruct((_B, _L, _G, _D), jnp.float32),
    mesh=plsc.VectorSubcoreMesh(core_axis_name="c", subcore_axis_name="s"),
    scratch_types=(
        [pltpu.VMEM((_L, _NP, _DH), jnp.float32)] * 8
        + [pltpu.SemaphoreType.DMA] * 8
    ),
)


def kernel(q, k, v):
    del q, k
    out = _sc_sort(v.reshape(_B, _L, _G, _D))
    return out.reshape(_B, _S, _D)
